# baseline (speedup 1.0000x reference)
import jax
import jax.numpy as jnp
from jax.experimental import pallas as pl
from jax.experimental.pallas import tpu as pltpu

N = 4096
D = 256
BM = 512
BK = 512


def _body(adj_ref, emb_ref, out_ref):
    k = pl.program_id(0)
    i = pl.program_id(1)
    acc = jnp.dot(adj_ref[...], emb_ref[...], preferred_element_type=jnp.float32)
    rows = pl.ds(i * BM, BM)

    @pl.when(k == 0)
    def _init():
        out_ref[rows, :] = acc

    @pl.when(k > 0)
    def _accum():
        out_ref[rows, :] = out_ref[rows, :] + acc


@jax.jit
def kernel(adj, embeds):
    return pl.pallas_call(
        _body,
        grid=(N // BK, N // BM),
        in_specs=[
            pl.BlockSpec((BM, BK), lambda k, i: (i, k)),
            pl.BlockSpec((BK, D), lambda k, i: (k, 0)),
        ],
        out_specs=pl.BlockSpec((N, D), lambda k, i: (0, 0)),
        out_shape=jax.ShapeDtypeStruct((N, D), jnp.float32),
        compiler_params=pltpu.CompilerParams(
            dimension_semantics=("arbitrary", "arbitrary"),
        ),
    )(adj, embeds)


# speedup vs baseline: 2.2435x; 2.2435x over previous
import jax
import jax.numpy as jnp
from jax.experimental import pallas as pl
from jax.experimental.pallas import tpu as pltpu

N = 4096
D = 256
BM = 512


def _body(adj_ref, emb_ref, out_ref):
    out_ref[...] = jnp.dot(
        adj_ref[...], emb_ref[...], preferred_element_type=jnp.float32
    )


@jax.jit
def kernel(adj, embeds):
    return pl.pallas_call(
        _body,
        grid=(N // BM,),
        in_specs=[
            pl.BlockSpec((BM, N), lambda i: (i, 0)),
            pl.BlockSpec((N, D), lambda i: (0, 0)),
        ],
        out_specs=pl.BlockSpec((BM, D), lambda i: (i, 0)),
        out_shape=jax.ShapeDtypeStruct((N, D), jnp.float32),
        compiler_params=pltpu.CompilerParams(
            dimension_semantics=("parallel",),
        ),
    )(adj, embeds)
